# Initial kernel scaffold; baseline (speedup 1.0000x reference)
import jax
import jax.numpy as jnp
from jax.experimental import pallas as pl


def kernel(x_paper, x_author, batch_paper, batch_author):
    raise NotImplementedError("write your pallas kernel here")


# speedup vs baseline: 2.8170x; 2.8170x over previous
import functools

import jax
import jax.numpy as jnp
from jax import lax
from jax.experimental import pallas as pl
from jax.experimental.pallas import tpu as pltpu
from jax.experimental.pallas import tpu_sc as plsc

N = 320000
D = 128
G = 1024
NC = 2
NS = 16
NW = NC * NS
ROWS_PER_TILE = N // NW
BLK = 80
NBLK = ROWS_PER_TILE // BLK
SLAB = G // NS
CW = 128


def _sc_body(xp, xa, bp, ba, sums, cnts,
             xbuf, idxbuf, onesbuf, zbuf, zcbuf, acc_p, acc_a, cnt_p, cnt_a):
    c = lax.axis_index("c")
    s = lax.axis_index("s")
    wid = c * NS + s
    row0 = wid * ROWS_PER_TILE

    zero16 = jnp.zeros((16,), jnp.float32)
    one16 = jnp.ones((16,), jnp.float32)

    def fill_z(r, _):
        for j in range(D // 16):
            zbuf[r, pl.ds(j * 16, 16)] = zero16
            zcbuf[r, pl.ds(j * 16, 16)] = zero16
        return 0

    def fill_one(r, _):
        for j in range(CW // 16):
            onesbuf[r, pl.ds(j * 16, 16)] = one16
        return 0

    lax.fori_loop(0, SLAB, fill_z, 0)
    lax.fori_loop(0, BLK, fill_one, 0)

    slab = pl.ds(s * SLAB, SLAB)
    pltpu.sync_copy(zbuf.at[pl.ds(0, SLAB)], acc_p.at[slab])
    pltpu.sync_copy(zbuf.at[pl.ds(0, SLAB)], acc_a.at[slab])
    pltpu.sync_copy(zcbuf, cnt_p.at[slab])
    pltpu.sync_copy(zcbuf, cnt_a.at[slab])
    plsc.subcore_barrier()

    def scatter_type(x_hbm, b_hbm, acc, cnt):
        def step(i, _):
            base = row0 + i * BLK
            pltpu.sync_copy(b_hbm.at[pl.ds(base, BLK)], idxbuf)
            pltpu.sync_copy(x_hbm.at[pl.ds(base, BLK)], xbuf)
            pltpu.sync_copy(xbuf, acc.at[idxbuf], add=True)
            pltpu.sync_copy(onesbuf, cnt.at[idxbuf], add=True)
            return 0

        lax.fori_loop(0, NBLK, step, 0)

    scatter_type(xp, bp, acc_p, cnt_p)
    scatter_type(xa, ba, acc_a, cnt_a)
    plsc.subcore_barrier()

    out0 = c * (2 * G) + s * SLAB
    pltpu.sync_copy(acc_p.at[slab], sums.at[pl.ds(out0, SLAB)])
    pltpu.sync_copy(acc_a.at[slab], sums.at[pl.ds(out0 + G, SLAB)])
    pltpu.sync_copy(cnt_p.at[slab], cnts.at[pl.ds(out0, SLAB)])
    pltpu.sync_copy(cnt_a.at[slab], cnts.at[pl.ds(out0 + G, SLAB)])


_sc_segsum = pl.kernel(
    _sc_body,
    out_type=(
        jax.ShapeDtypeStruct((2 * NC * G, D), jnp.float32),
        jax.ShapeDtypeStruct((2 * NC * G, CW), jnp.float32),
    ),
    mesh=plsc.VectorSubcoreMesh(core_axis_name="c", subcore_axis_name="s"),
    scratch_types=[
        pltpu.VMEM((BLK, D), jnp.float32),
        pltpu.VMEM((BLK,), jnp.int32),
        pltpu.VMEM((BLK, CW), jnp.float32),
        pltpu.VMEM((SLAB, D), jnp.float32),
        pltpu.VMEM((SLAB, CW), jnp.float32),
        pltpu.VMEM_SHARED((G, D), jnp.float32),
        pltpu.VMEM_SHARED((G, D), jnp.float32),
        pltpu.VMEM_SHARED((G, CW), jnp.float32),
        pltpu.VMEM_SHARED((G, CW), jnp.float32),
    ],
)


def _finalize_body(sums_ref, cnts_ref, o_ref):
    sp = sums_ref[0:G, :] + sums_ref[2 * G:3 * G, :]
    sa = sums_ref[G:2 * G, :] + sums_ref[3 * G:4 * G, :]
    cp = cnts_ref[0:G, 0:1] + cnts_ref[2 * G:3 * G, 0:1]
    ca = cnts_ref[G:2 * G, 0:1] + cnts_ref[3 * G:4 * G, 0:1]
    o_ref[...] = sp / jnp.maximum(cp, 1.0) + sa / jnp.maximum(ca, 1.0)


_tc_finalize = pl.pallas_call(
    _finalize_body,
    out_shape=jax.ShapeDtypeStruct((G, D), jnp.float32),
)


@jax.jit
def kernel(x_paper, x_author, batch_paper, batch_author):
    sums, cnts = _sc_segsum(x_paper, x_author, batch_paper, batch_author)
    return _tc_finalize(sums, cnts)


# speedup vs baseline: 4.0614x; 1.4417x over previous
import jax
import jax.numpy as jnp
from jax import lax
from jax.experimental import pallas as pl
from jax.experimental.pallas import tpu as pltpu
from jax.experimental.pallas import tpu_sc as plsc

N = 320000
D = 128
G = 1024
NC = 2
NS = 16
NW = NC * NS
ROWS_PER_TILE = N // NW
BLK = 80
NBLK = ROWS_PER_TILE // BLK
SLAB = G // NS
CW = 128


def _sc_body(xp, xa, bp, ba, sums, cnts,
             xb0, xb1, ib0, ib1, onesbuf, zcbuf, acc_p, acc_a, cnt_p, cnt_a,
             sem0, sem1):
    c = lax.axis_index("c")
    s = lax.axis_index("s")
    wid = c * NS + s
    row0 = wid * ROWS_PER_TILE

    zero16 = jnp.zeros((16,), jnp.float32)
    one16 = jnp.ones((16,), jnp.float32)

    def fill_z(r, _):
        for j in range(D // 16):
            xb0[r, pl.ds(j * 16, 16)] = zero16
        for j in range(CW // 16):
            zcbuf[r, pl.ds(j * 16, 16)] = zero16
        return 0

    def fill_one(r, _):
        for j in range(CW // 16):
            onesbuf[r, pl.ds(j * 16, 16)] = one16
        return 0

    lax.fori_loop(0, SLAB, fill_z, 0)
    lax.fori_loop(0, BLK, fill_one, 0)

    slab = pl.ds(s * SLAB, SLAB)
    zsrc = xb0.at[pl.ds(0, SLAB)]
    pltpu.sync_copy(zsrc, acc_p.at[slab])
    pltpu.sync_copy(zsrc, acc_a.at[slab])
    pltpu.sync_copy(zcbuf, cnt_p.at[slab])
    pltpu.sync_copy(zcbuf, cnt_a.at[slab])
    plsc.subcore_barrier()

    def gather(x_hbm, b_hbm, blk, xb, ib):
        base = row0 + blk * BLK
        pltpu.sync_copy(b_hbm.at[pl.ds(base, BLK)], ib)
        pltpu.sync_copy(x_hbm.at[pl.ds(base, BLK)], xb)

    def fire(acc, cnt, xb, ib, sem):
        pltpu.async_copy(xb, acc.at[ib], sem, add=True)
        pltpu.async_copy(onesbuf, cnt.at[ib], sem, add=True)

    def drain(acc, cnt, xb, ib, sem):
        pltpu.make_async_copy(xb, acc.at[ib], sem).wait()
        pltpu.make_async_copy(onesbuf, cnt.at[ib], sem).wait()

    def scatter_type(x_hbm, b_hbm, acc, cnt):
        gather(x_hbm, b_hbm, 0, xb0, ib0)
        fire(acc, cnt, xb0, ib0, sem0)

        def step(p, _):
            @pl.when(p > 0)
            def _():
                drain(acc, cnt, xb1, ib1, sem1)

            gather(x_hbm, b_hbm, 2 * p + 1, xb1, ib1)
            fire(acc, cnt, xb1, ib1, sem1)
            drain(acc, cnt, xb0, ib0, sem0)
            gather(x_hbm, b_hbm, 2 * p + 2, xb0, ib0)
            fire(acc, cnt, xb0, ib0, sem0)
            return 0

        lax.fori_loop(0, (NBLK - 1) // 2, step, 0)
        drain(acc, cnt, xb0, ib0, sem0)
        drain(acc, cnt, xb1, ib1, sem1)

    scatter_type(xp, bp, acc_p, cnt_p)
    scatter_type(xa, ba, acc_a, cnt_a)
    plsc.subcore_barrier()

    out0 = c * (2 * G) + s * SLAB
    pltpu.sync_copy(acc_p.at[slab], sums.at[pl.ds(out0, SLAB)])
    pltpu.sync_copy(acc_a.at[slab], sums.at[pl.ds(out0 + G, SLAB)])
    pltpu.sync_copy(cnt_p.at[slab], cnts.at[pl.ds(out0, SLAB)])
    pltpu.sync_copy(cnt_a.at[slab], cnts.at[pl.ds(out0 + G, SLAB)])


_sc_segsum = pl.kernel(
    _sc_body,
    out_type=(
        jax.ShapeDtypeStruct((2 * NC * G, D), jnp.float32),
        jax.ShapeDtypeStruct((2 * NC * G, CW), jnp.float32),
    ),
    mesh=plsc.VectorSubcoreMesh(core_axis_name="c", subcore_axis_name="s"),
    scratch_types=[
        pltpu.VMEM((BLK, D), jnp.float32),
        pltpu.VMEM((BLK, D), jnp.float32),
        pltpu.VMEM((BLK,), jnp.int32),
        pltpu.VMEM((BLK,), jnp.int32),
        pltpu.VMEM((BLK, CW), jnp.float32),
        pltpu.VMEM((SLAB, CW), jnp.float32),
        pltpu.VMEM_SHARED((G, D), jnp.float32),
        pltpu.VMEM_SHARED((G, D), jnp.float32),
        pltpu.VMEM_SHARED((G, CW), jnp.float32),
        pltpu.VMEM_SHARED((G, CW), jnp.float32),
        pltpu.SemaphoreType.DMA,
        pltpu.SemaphoreType.DMA,
    ],
)


def _finalize_body(sums_ref, cnts_ref, o_ref):
    sp = sums_ref[0:G, :] + sums_ref[2 * G:3 * G, :]
    sa = sums_ref[G:2 * G, :] + sums_ref[3 * G:4 * G, :]
    cp = cnts_ref[0:G, 0:1] + cnts_ref[2 * G:3 * G, 0:1]
    ca = cnts_ref[G:2 * G, 0:1] + cnts_ref[3 * G:4 * G, 0:1]
    o_ref[...] = sp / jnp.maximum(cp, 1.0) + sa / jnp.maximum(ca, 1.0)


_tc_finalize = pl.pallas_call(
    _finalize_body,
    out_shape=jax.ShapeDtypeStruct((G, D), jnp.float32),
)


@jax.jit
def kernel(x_paper, x_author, batch_paper, batch_author):
    sums, cnts = _sc_segsum(x_paper, x_author, batch_paper, batch_author)
    return _tc_finalize(sums, cnts)


# speedup vs baseline: 4.1857x; 1.0306x over previous
import jax
import jax.numpy as jnp
from jax import lax
from jax.experimental import pallas as pl
from jax.experimental.pallas import tpu as pltpu
from jax.experimental.pallas import tpu_sc as plsc

N = 320000
D = 128
G = 1024
NC = 2
NS = 16
NW = NC * NS
ROWS_PER_TILE = N // NW
BLK = 80
NBLK = ROWS_PER_TILE // BLK
SLAB = G // NS
CW = 128
TRASH = G


def _sc_body(xp, xa, bp, ba, sums, cnts,
             xb0, xb1, ib0, ib1, ibf0, ibf1, onesbuf, blkbuf, zcbuf,
             acc_p, acc_a, cnt_p, cnt_a, sem0, sem1):
    c = lax.axis_index("c")
    s = lax.axis_index("s")
    wid = c * NS + s
    row0 = wid * ROWS_PER_TILE

    zero16 = jnp.zeros((16,), jnp.float32)
    one16 = jnp.ones((16,), jnp.float32)

    def fill_z(r, _):
        for j in range(D // 16):
            xb0[r, pl.ds(j * 16, 16)] = zero16
        for j in range(CW // 16):
            zcbuf[r, pl.ds(j * 16, 16)] = zero16
        return 0

    def fill_one(r, _):
        for j in range(CW // 16):
            onesbuf[r, pl.ds(j * 16, 16)] = one16
        return 0

    lax.fori_loop(0, SLAB, fill_z, 0)
    lax.fori_loop(0, BLK, fill_one, 0)

    blk16 = jnp.full((16,), float(BLK), jnp.float32)
    for j in range(CW // 16):
        blkbuf[0, pl.ds(j * 16, 16)] = blk16
    def fill_blkz(r, _):
        for j in range(CW // 16):
            blkbuf[r, pl.ds(j * 16, 16)] = zero16
        return 0
    lax.fori_loop(1, 16, fill_blkz, 0)

    slab = pl.ds(s * SLAB, SLAB)
    zsrc = xb0.at[pl.ds(0, SLAB)]
    pltpu.sync_copy(zsrc, acc_p.at[slab])
    pltpu.sync_copy(zsrc, acc_a.at[slab])
    pltpu.sync_copy(zcbuf, cnt_p.at[slab])
    pltpu.sync_copy(zcbuf, cnt_a.at[slab])
    plsc.subcore_barrier()

    lane = lax.broadcasted_iota(jnp.int32, (16,), 0)

    def gather(x_hbm, b_hbm, blk, xb, ib, ibf):
        base = row0 + blk * BLK
        pltpu.sync_copy(b_hbm.at[pl.ds(base, BLK)], ib)
        pltpu.sync_copy(x_hbm.at[pl.ds(base, BLK)], xb)
        head = ib[pl.ds(0, 16)]
        ibf[...] = jnp.where(lane == 0, head, TRASH)

    def single_run(ib):
        head = ib[pl.ds(0, 16)]
        tail = ib[pl.ds(BLK - 16, 16)]
        return head[0] == tail[15]

    def fire(acc, cnt, xb, ib, ibf, sem):
        pltpu.async_copy(xb, acc.at[ib], sem, add=True)

        @pl.when(single_run(ib))
        def _():
            pltpu.async_copy(blkbuf, cnt.at[ibf], sem, add=True)

        @pl.when(jnp.logical_not(single_run(ib)))
        def _():
            pltpu.async_copy(onesbuf, cnt.at[ib], sem, add=True)

    def drain(acc, cnt, xb, ib, ibf, sem):
        pltpu.make_async_copy(xb, acc.at[ib], sem).wait()

        @pl.when(single_run(ib))
        def _():
            pltpu.make_async_copy(blkbuf, cnt.at[ibf], sem).wait()

        @pl.when(jnp.logical_not(single_run(ib)))
        def _():
            pltpu.make_async_copy(onesbuf, cnt.at[ib], sem).wait()

    def scatter_type(x_hbm, b_hbm, acc, cnt):
        gather(x_hbm, b_hbm, 0, xb0, ib0, ibf0)
        fire(acc, cnt, xb0, ib0, ibf0, sem0)

        def step(p, _):
            @pl.when(p > 0)
            def _():
                drain(acc, cnt, xb1, ib1, ibf1, sem1)

            gather(x_hbm, b_hbm, 2 * p + 1, xb1, ib1, ibf1)
            fire(acc, cnt, xb1, ib1, ibf1, sem1)
            drain(acc, cnt, xb0, ib0, ibf0, sem0)
            gather(x_hbm, b_hbm, 2 * p + 2, xb0, ib0, ibf0)
            fire(acc, cnt, xb0, ib0, ibf0, sem0)
            return 0

        lax.fori_loop(0, (NBLK - 1) // 2, step, 0)
        drain(acc, cnt, xb0, ib0, ibf0, sem0)
        drain(acc, cnt, xb1, ib1, ibf1, sem1)

    scatter_type(xp, bp, acc_p, cnt_p)
    scatter_type(xa, ba, acc_a, cnt_a)
    plsc.subcore_barrier()

    out0 = c * (2 * G) + s * SLAB
    pltpu.sync_copy(acc_p.at[slab], sums.at[pl.ds(out0, SLAB)])
    pltpu.sync_copy(acc_a.at[slab], sums.at[pl.ds(out0 + G, SLAB)])
    pltpu.sync_copy(cnt_p.at[slab], cnts.at[pl.ds(out0, SLAB)])
    pltpu.sync_copy(cnt_a.at[slab], cnts.at[pl.ds(out0 + G, SLAB)])


_sc_segsum = pl.kernel(
    _sc_body,
    out_type=(
        jax.ShapeDtypeStruct((2 * NC * G, D), jnp.float32),
        jax.ShapeDtypeStruct((2 * NC * G, CW), jnp.float32),
    ),
    mesh=plsc.VectorSubcoreMesh(core_axis_name="c", subcore_axis_name="s"),
    scratch_types=[
        pltpu.VMEM((BLK, D), jnp.float32),
        pltpu.VMEM((BLK, D), jnp.float32),
        pltpu.VMEM((BLK,), jnp.int32),
        pltpu.VMEM((BLK,), jnp.int32),
        pltpu.VMEM((16,), jnp.int32),
        pltpu.VMEM((16,), jnp.int32),
        pltpu.VMEM((BLK, CW), jnp.float32),
        pltpu.VMEM((16, CW), jnp.float32),
        pltpu.VMEM((SLAB, CW), jnp.float32),
        pltpu.VMEM_SHARED((G, D), jnp.float32),
        pltpu.VMEM_SHARED((G, D), jnp.float32),
        pltpu.VMEM_SHARED((G + 16, CW), jnp.float32),
        pltpu.VMEM_SHARED((G + 16, CW), jnp.float32),
        pltpu.SemaphoreType.DMA,
        pltpu.SemaphoreType.DMA,
    ],
)


def _finalize_body(sums_ref, cnts_ref, o_ref):
    sp = sums_ref[0:G, :] + sums_ref[2 * G:3 * G, :]
    sa = sums_ref[G:2 * G, :] + sums_ref[3 * G:4 * G, :]
    cp = cnts_ref[0:G, 0:1] + cnts_ref[2 * G:3 * G, 0:1]
    ca = cnts_ref[G:2 * G, 0:1] + cnts_ref[3 * G:4 * G, 0:1]
    o_ref[...] = sp / jnp.maximum(cp, 1.0) + sa / jnp.maximum(ca, 1.0)


_tc_finalize = pl.pallas_call(
    _finalize_body,
    out_shape=jax.ShapeDtypeStruct((G, D), jnp.float32),
)


@jax.jit
def kernel(x_paper, x_author, batch_paper, batch_author):
    sums, cnts = _sc_segsum(x_paper, x_author, batch_paper, batch_author)
    return _tc_finalize(sums, cnts)


# speedup vs baseline: 5.8743x; 1.4034x over previous
import jax
import jax.numpy as jnp
from jax import lax
from jax.experimental import pallas as pl
from jax.experimental.pallas import tpu as pltpu
from jax.experimental.pallas import tpu_sc as plsc

N = 320000
D = 128
G = 1024
NC = 2
NS = 16
NW = NC * NS
ROWS_PER_TILE = N // NW
BLK = 80
NBLK = ROWS_PER_TILE // BLK
NBUF = 5
LA = 3
SLAB = G // NS
CW = 128
TRASH = G


def _sc_body(xp, xa, bp, ba, sums, cnts,
             xbs, ibs, ibfs, onesbuf, blkbuf,
             acc_p, acc_a, cnt_p, cnt_a, semg, sems):
    c = lax.axis_index("c")
    s = lax.axis_index("s")
    wid = c * NS + s
    row0 = wid * ROWS_PER_TILE

    zero16 = jnp.zeros((16,), jnp.float32)
    one16 = jnp.ones((16,), jnp.float32)

    def fill_z(r, _):
        for j in range(D // 16):
            xbs[0][r, pl.ds(j * 16, 16)] = zero16
        return 0

    def fill_one(r, _):
        for j in range(CW // 16):
            onesbuf[r, pl.ds(j * 16, 16)] = one16
        return 0

    lax.fori_loop(0, SLAB, fill_z, 0)
    lax.fori_loop(0, BLK, fill_one, 0)

    blk16 = jnp.full((16,), float(BLK), jnp.float32)
    for j in range(CW // 16):
        blkbuf[0, pl.ds(j * 16, 16)] = blk16

    def fill_blkz(r, _):
        for j in range(CW // 16):
            blkbuf[r, pl.ds(j * 16, 16)] = zero16
        return 0

    lax.fori_loop(1, 16, fill_blkz, 0)

    slab = pl.ds(s * SLAB, SLAB)
    zsrc = xbs[0].at[pl.ds(0, SLAB)]
    pltpu.sync_copy(zsrc, acc_p.at[slab])
    pltpu.sync_copy(zsrc, acc_a.at[slab])
    pltpu.sync_copy(zsrc, cnt_p.at[slab])
    pltpu.sync_copy(zsrc, cnt_a.at[slab])
    plsc.subcore_barrier()

    lane = lax.broadcasted_iota(jnp.int32, (16,), 0)

    def fire_gather(x_hbm, b_hbm, blk, b):
        base = row0 + blk * BLK
        pltpu.async_copy(b_hbm.at[pl.ds(base, BLK)], ibs[b], semg[b])
        pltpu.async_copy(x_hbm.at[pl.ds(base, BLK)], xbs[b], semg[b])

    def drain_gather(x_hbm, b_hbm, blk, b):
        base = row0 + blk * BLK
        pltpu.make_async_copy(b_hbm.at[pl.ds(base, BLK)], ibs[b], semg[b]).wait()
        pltpu.make_async_copy(x_hbm.at[pl.ds(base, BLK)], xbs[b], semg[b]).wait()

    def single_run(b):
        head = ibs[b][pl.ds(0, 16)]
        tail = ibs[b][pl.ds(BLK - 16, 16)]
        return head[0] == tail[15]

    def fire_scatter(acc, cnt, b):
        head = ibs[b][pl.ds(0, 16)]
        ibfs[b][...] = jnp.where(lane == 0, head, TRASH)
        pltpu.async_copy(xbs[b], acc.at[ibs[b]], sems[b], add=True)

        @pl.when(single_run(b))
        def _():
            pltpu.async_copy(blkbuf, cnt.at[ibfs[b]], sems[b], add=True)

        @pl.when(jnp.logical_not(single_run(b)))
        def _():
            pltpu.async_copy(onesbuf, cnt.at[ibs[b]], sems[b], add=True)

    def drain_scatter(acc, cnt, b):
        pltpu.make_async_copy(xbs[b], acc.at[ibs[b]], sems[b]).wait()

        @pl.when(single_run(b))
        def _():
            pltpu.make_async_copy(blkbuf, cnt.at[ibfs[b]], sems[b]).wait()

        @pl.when(jnp.logical_not(single_run(b)))
        def _():
            pltpu.make_async_copy(onesbuf, cnt.at[ibs[b]], sems[b]).wait()

    def scatter_type(x_hbm, b_hbm, acc, cnt):
        for i in range(LA):
            fire_gather(x_hbm, b_hbm, i, i)

        def step(p, _):
            for q in range(NBUF):
                i = NBUF * p + q
                drain_gather(x_hbm, b_hbm, i, q)
                fire_scatter(acc, cnt, q)
                nxt = i + LA
                bn = (q + LA) % NBUF

                @pl.when(nxt < NBLK)
                def _():
                    @pl.when(nxt >= NBUF)
                    def _():
                        drain_scatter(acc, cnt, bn)

                    fire_gather(x_hbm, b_hbm, nxt, bn)
            return 0

        lax.fori_loop(0, NBLK // NBUF, step, 0)
        for q in range(NBUF):
            drain_scatter(acc, cnt, q)

    scatter_type(xp, bp, acc_p, cnt_p)
    scatter_type(xa, ba, acc_a, cnt_a)
    plsc.subcore_barrier()

    out0 = c * (2 * G) + s * SLAB
    pltpu.sync_copy(acc_p.at[slab], sums.at[pl.ds(out0, SLAB)])
    pltpu.sync_copy(acc_a.at[slab], sums.at[pl.ds(out0 + G, SLAB)])
    pltpu.sync_copy(cnt_p.at[slab], cnts.at[pl.ds(out0, SLAB)])
    pltpu.sync_copy(cnt_a.at[slab], cnts.at[pl.ds(out0 + G, SLAB)])


_sc_segsum = pl.kernel(
    _sc_body,
    out_type=(
        jax.ShapeDtypeStruct((2 * NC * G, D), jnp.float32),
        jax.ShapeDtypeStruct((2 * NC * G, CW), jnp.float32),
    ),
    mesh=plsc.VectorSubcoreMesh(core_axis_name="c", subcore_axis_name="s"),
    scratch_types=[
        [pltpu.VMEM((BLK, D), jnp.float32) for _ in range(NBUF)],
        [pltpu.VMEM((BLK,), jnp.int32) for _ in range(NBUF)],
        [pltpu.VMEM((16,), jnp.int32) for _ in range(NBUF)],
        pltpu.VMEM((BLK, CW), jnp.float32),
        pltpu.VMEM((16, CW), jnp.float32),
        pltpu.VMEM_SHARED((G, D), jnp.float32),
        pltpu.VMEM_SHARED((G, D), jnp.float32),
        pltpu.VMEM_SHARED((G + 16, CW), jnp.float32),
        pltpu.VMEM_SHARED((G + 16, CW), jnp.float32),
        [pltpu.SemaphoreType.DMA for _ in range(NBUF)],
        [pltpu.SemaphoreType.DMA for _ in range(NBUF)],
    ],
)


def _finalize_body(sums_ref, cnts_ref, o_ref):
    sp = sums_ref[0:G, :] + sums_ref[2 * G:3 * G, :]
    sa = sums_ref[G:2 * G, :] + sums_ref[3 * G:4 * G, :]
    cp = cnts_ref[0:G, 0:1] + cnts_ref[2 * G:3 * G, 0:1]
    ca = cnts_ref[G:2 * G, 0:1] + cnts_ref[3 * G:4 * G, 0:1]
    o_ref[...] = sp / jnp.maximum(cp, 1.0) + sa / jnp.maximum(ca, 1.0)


_tc_finalize = pl.pallas_call(
    _finalize_body,
    out_shape=jax.ShapeDtypeStruct((G, D), jnp.float32),
)


@jax.jit
def kernel(x_paper, x_author, batch_paper, batch_author):
    sums, cnts = _sc_segsum(x_paper, x_author, batch_paper, batch_author)
    return _tc_finalize(sums, cnts)


# speedup vs baseline: 7.2572x; 1.2354x over previous
import jax
import jax.numpy as jnp
from jax import lax
from jax.experimental import pallas as pl
from jax.experimental.pallas import tpu as pltpu
from jax.experimental.pallas import tpu_sc as plsc

N = 320000
D = 128
G = 1024
NC = 2
NS = 16
NW = NC * NS
ROWS_PER_TILE = N // NW
BLK = 80
NBLK = ROWS_PER_TILE // BLK
NBUF = 5
LA = 3
SLAB = G // NS
CW = 128
TRASH = G
FP = 16


def _sc_body(xp, xa, bp, ba, sums, cnts,
             xbs, ibs, ibfs, onesbuf, blkbuf,
             acc_p, acc_a, cnt_p, cnt_a, semg, sems):
    c = lax.axis_index("c")
    s = lax.axis_index("s")
    wid = c * NS + s
    row0 = wid * ROWS_PER_TILE

    zero16 = jnp.zeros((16,), jnp.float32)
    one16 = jnp.ones((16,), jnp.float32)

    def fill_z(r, _):
        for j in range(D // 16):
            xbs[0][r, pl.ds(j * 16, 16)] = zero16
        return 0

    def fill_one(r, _):
        for j in range(CW // 16):
            onesbuf[r, pl.ds(j * 16, 16)] = one16
        return 0

    lax.fori_loop(0, SLAB, fill_z, 0)
    lax.fori_loop(0, BLK, fill_one, 0)

    blk16 = jnp.full((16,), float(BLK), jnp.float32)
    for j in range(CW // 16):
        blkbuf[0, pl.ds(j * 16, 16)] = blk16

    def fill_blkz(r, _):
        for j in range(CW // 16):
            blkbuf[r, pl.ds(j * 16, 16)] = zero16
        return 0

    lax.fori_loop(1, FP, fill_blkz, 0)

    slab = pl.ds(s * SLAB, SLAB)
    zsrc = xbs[0].at[pl.ds(0, SLAB)]
    pltpu.sync_copy(zsrc, acc_p.at[slab])
    pltpu.sync_copy(zsrc, acc_a.at[slab])
    pltpu.sync_copy(zsrc, cnt_p.at[slab])
    pltpu.sync_copy(zsrc, cnt_a.at[slab])
    plsc.subcore_barrier()

    lane = lax.broadcasted_iota(jnp.int32, (16,), 0)

    def fire_gather(x_hbm, b_hbm, blk, b):
        base = row0 + blk * BLK
        pltpu.async_copy(b_hbm.at[pl.ds(base, BLK)], ibs[b], semg[b])
        pltpu.async_copy(x_hbm.at[pl.ds(base, BLK)], xbs[b], semg[b])

    def drain_gather(x_hbm, b_hbm, blk, b):
        base = row0 + blk * BLK
        pltpu.make_async_copy(b_hbm.at[pl.ds(base, BLK)], ibs[b], semg[b]).wait()
        pltpu.make_async_copy(x_hbm.at[pl.ds(base, BLK)], xbs[b], semg[b]).wait()

    def single_run(b):
        head = ibs[b][pl.ds(0, 16)]
        tail = ibs[b][pl.ds(BLK - 16, 16)]
        return head[0] == tail[15]

    def fire_scatter(acc, cnt, b):
        head = ibs[b][pl.ds(0, 16)]
        ibfs[b][...] = jnp.where(lane == 0, head, TRASH)

        @pl.when(single_run(b))
        def _():
            def srow(r, carry):
                return tuple(carry[j] + xbs[b][r, pl.ds(j * 16, 16)]
                             for j in range(D // 16))

            tot = lax.fori_loop(0, BLK, srow,
                                tuple(jnp.zeros((16,), jnp.float32)
                                      for _ in range(D // 16)))
            for j in range(D // 16):
                xbs[b][0, pl.ds(j * 16, 16)] = tot[j]
            pltpu.async_copy(xbs[b].at[pl.ds(0, 16)], acc.at[ibfs[b]],
                             sems[b], add=True)
            pltpu.async_copy(blkbuf, cnt.at[ibfs[b]], sems[b], add=True)

        @pl.when(jnp.logical_not(single_run(b)))
        def _():
            pltpu.async_copy(xbs[b], acc.at[ibs[b]], sems[b], add=True)
            pltpu.async_copy(onesbuf, cnt.at[ibs[b]], sems[b], add=True)

    def drain_scatter(acc, cnt, b):
        @pl.when(single_run(b))
        def _():
            pltpu.make_async_copy(xbs[b].at[pl.ds(0, 16)], acc.at[ibfs[b]],
                                  sems[b]).wait()
            pltpu.make_async_copy(blkbuf, cnt.at[ibfs[b]], sems[b]).wait()

        @pl.when(jnp.logical_not(single_run(b)))
        def _():
            pltpu.make_async_copy(xbs[b], acc.at[ibs[b]], sems[b]).wait()
            pltpu.make_async_copy(onesbuf, cnt.at[ibs[b]], sems[b]).wait()

    def scatter_type(x_hbm, b_hbm, acc, cnt):
        for i in range(LA):
            fire_gather(x_hbm, b_hbm, i, i)

        def step(p, _):
            for q in range(NBUF):
                i = NBUF * p + q
                drain_gather(x_hbm, b_hbm, i, q)
                fire_scatter(acc, cnt, q)
                nxt = i + LA
                bn = (q + LA) % NBUF

                @pl.when(nxt < NBLK)
                def _():
                    @pl.when(nxt >= NBUF)
                    def _():
                        drain_scatter(acc, cnt, bn)

                    fire_gather(x_hbm, b_hbm, nxt, bn)
            return 0

        lax.fori_loop(0, NBLK // NBUF, step, 0)
        for q in range(NBUF):
            drain_scatter(acc, cnt, q)

    scatter_type(xp, bp, acc_p, cnt_p)
    scatter_type(xa, ba, acc_a, cnt_a)
    plsc.subcore_barrier()

    out0 = c * (2 * G) + s * SLAB
    pltpu.sync_copy(acc_p.at[slab], sums.at[pl.ds(out0, SLAB)])
    pltpu.sync_copy(acc_a.at[slab], sums.at[pl.ds(out0 + G, SLAB)])
    pltpu.sync_copy(cnt_p.at[slab], cnts.at[pl.ds(out0, SLAB)])
    pltpu.sync_copy(cnt_a.at[slab], cnts.at[pl.ds(out0 + G, SLAB)])


_sc_segsum = pl.kernel(
    _sc_body,
    out_type=(
        jax.ShapeDtypeStruct((2 * NC * G, D), jnp.float32),
        jax.ShapeDtypeStruct((2 * NC * G, CW), jnp.float32),
    ),
    mesh=plsc.VectorSubcoreMesh(core_axis_name="c", subcore_axis_name="s"),
    scratch_types=[
        [pltpu.VMEM((BLK, D), jnp.float32) for _ in range(NBUF)],
        [pltpu.VMEM((BLK,), jnp.int32) for _ in range(NBUF)],
        [pltpu.VMEM((16,), jnp.int32) for _ in range(NBUF)],
        pltpu.VMEM((BLK, CW), jnp.float32),
        pltpu.VMEM((16, CW), jnp.float32),
        pltpu.VMEM_SHARED((G + 16, D), jnp.float32),
        pltpu.VMEM_SHARED((G + 16, D), jnp.float32),
        pltpu.VMEM_SHARED((G + 16, CW), jnp.float32),
        pltpu.VMEM_SHARED((G + 16, CW), jnp.float32),
        [pltpu.SemaphoreType.DMA for _ in range(NBUF)],
        [pltpu.SemaphoreType.DMA for _ in range(NBUF)],
    ],
)


def _finalize_body(sums_ref, cnts_ref, o_ref):
    sp = sums_ref[0:G, :] + sums_ref[2 * G:3 * G, :]
    sa = sums_ref[G:2 * G, :] + sums_ref[3 * G:4 * G, :]
    cp = cnts_ref[0:G, 0:1] + cnts_ref[2 * G:3 * G, 0:1]
    ca = cnts_ref[G:2 * G, 0:1] + cnts_ref[3 * G:4 * G, 0:1]
    o_ref[...] = sp / jnp.maximum(cp, 1.0) + sa / jnp.maximum(ca, 1.0)


_tc_finalize = pl.pallas_call(
    _finalize_body,
    out_shape=jax.ShapeDtypeStruct((G, D), jnp.float32),
)


@jax.jit
def kernel(x_paper, x_author, batch_paper, batch_author):
    sums, cnts = _sc_segsum(x_paper, x_author, batch_paper, batch_author)
    return _tc_finalize(sums, cnts)
